# Initial kernel scaffold; baseline (speedup 1.0000x reference)
import jax
import jax.numpy as jnp
from jax.experimental import pallas as pl


def kernel(user_feat, biz_feat, adj_ub, adj_bu, adj_uu, adj_bb, W_user, W_biz):
    raise NotImplementedError("write your pallas kernel here")


# speedup vs baseline: 1.0694x; 1.0694x over previous
import functools

import jax
import jax.numpy as jnp
from jax.experimental import pallas as pl
from jax.experimental.pallas import tpu as pltpu

N = 4096
D = 128
IN_DIM = 384
BM = 512
BK = 512
NI = N // BM
NK = N // BK
EPS = 1e-12


def _dot(a, b):
    return jax.lax.dot_general(
        a, b, (((1,), (0,)), ((), ())), preferred_element_type=jnp.float32
    )


def _proj_kernel(uf, bf, wu, wb, u0, b0):
    u0[...] = _dot(uf[...], wu[...])
    b0[...] = _dot(bf[...], wb[...])


def _prop1_kernel(abu, auu, aub, abb, u, b, u0b, b0b, u1, b1, su, sb):
    k = pl.program_id(1)

    @pl.when(k == 0)
    def _():
        u1[...] = jnp.zeros_like(u1)
        b1[...] = jnp.zeros_like(b1)

    uk = u[pl.ds(k * BK, BK), :]
    bk = b[pl.ds(k * BK, BK), :]
    u1[...] += _dot(abu[...], bk) + _dot(auu[...], uk)
    b1[...] += _dot(aub[...], uk) + _dot(abb[...], bk)

    @pl.when(k == NK - 1)
    def _():
        su[...] = u0b[...] + u1[...]
        sb[...] = b0b[...] + b1[...]


def _prop2_kernel(abu, auu, aub, abb, u, b, sub, sbb, uh, bh, accu, accb):
    k = pl.program_id(1)

    @pl.when(k == 0)
    def _():
        accu[...] = jnp.zeros_like(accu)
        accb[...] = jnp.zeros_like(accb)

    uk = u[pl.ds(k * BK, BK), :]
    bk = b[pl.ds(k * BK, BK), :]
    accu[...] += _dot(abu[...], bk) + _dot(auu[...], uk)
    accb[...] += _dot(aub[...], uk) + _dot(abb[...], bk)

    @pl.when(k == NK - 1)
    def _():
        emb_u = (sub[...] + accu[...]) * (1.0 / 3.0)
        emb_b = (sbb[...] + accb[...]) * (1.0 / 3.0)
        nu = jnp.sqrt(jnp.sum(emb_u * emb_u, axis=-1, keepdims=True))
        nb = jnp.sqrt(jnp.sum(emb_b * emb_b, axis=-1, keepdims=True))
        uh[...] = emb_u / jnp.maximum(nu, EPS)
        bh[...] = emb_b / jnp.maximum(nb, EPS)


def _adj_spec():
    return pl.BlockSpec((BM, BK), lambda i, k: (i, k))


def _resident_spec():
    return pl.BlockSpec((N, D), lambda i, k: (0, 0))


def _row_spec():
    return pl.BlockSpec((BM, D), lambda i, k: (i, 0))


@functools.partial(jax.jit)
def kernel(user_feat, biz_feat, adj_ub, adj_bu, adj_uu, adj_bb, W_user, W_biz):
    emb = jax.ShapeDtypeStruct((N, D), jnp.float32)
    blk = jax.ShapeDtypeStruct((BM, D), jnp.float32)

    u0, b0 = pl.pallas_call(
        _proj_kernel,
        grid=(NI,),
        in_specs=[
            pl.BlockSpec((BM, IN_DIM), lambda i: (i, 0)),
            pl.BlockSpec((BM, IN_DIM), lambda i: (i, 0)),
            pl.BlockSpec((IN_DIM, D), lambda i: (0, 0)),
            pl.BlockSpec((IN_DIM, D), lambda i: (0, 0)),
        ],
        out_specs=[
            pl.BlockSpec((BM, D), lambda i: (i, 0)),
            pl.BlockSpec((BM, D), lambda i: (i, 0)),
        ],
        out_shape=[emb, emb],
        compiler_params=pltpu.CompilerParams(
            dimension_semantics=("parallel",),
        ),
    )(user_feat, biz_feat, W_user, W_biz)

    u1, b1, su, sb = pl.pallas_call(
        _prop1_kernel,
        grid=(NI, NK),
        in_specs=[
            _adj_spec(), _adj_spec(), _adj_spec(), _adj_spec(),
            _resident_spec(), _resident_spec(),
            _row_spec(), _row_spec(),
        ],
        out_specs=[_row_spec(), _row_spec(), _row_spec(), _row_spec()],
        out_shape=[emb, emb, emb, emb],
        compiler_params=pltpu.CompilerParams(
            dimension_semantics=("parallel", "arbitrary"),
        ),
    )(adj_bu, adj_uu, adj_ub, adj_bb, u0, b0, u0, b0)

    user_h, biz_h = pl.pallas_call(
        _prop2_kernel,
        grid=(NI, NK),
        in_specs=[
            _adj_spec(), _adj_spec(), _adj_spec(), _adj_spec(),
            _resident_spec(), _resident_spec(),
            _row_spec(), _row_spec(),
        ],
        out_specs=[_row_spec(), _row_spec()],
        out_shape=[emb, emb],
        scratch_shapes=[
            pltpu.VMEM((BM, D), jnp.float32),
            pltpu.VMEM((BM, D), jnp.float32),
        ],
        compiler_params=pltpu.CompilerParams(
            dimension_semantics=("parallel", "arbitrary"),
        ),
    )(adj_bu, adj_uu, adj_ub, adj_bb, u1, b1, su, sb)

    return (user_h, biz_h)
